# baseline (speedup 1.0000x reference)
import functools

import jax
import jax.numpy as jnp
from jax import lax
from jax.experimental import pallas as pl
from jax.experimental.pallas import tpu as pltpu
from jax.experimental.pallas import tpu_sc as plsc

NUM_CORES = 2
NUM_SUBCORES = 16
NUM_WORKERS = NUM_CORES * NUM_SUBCORES
BATCH = 16384
EMBED_DIM = 64
NUM_ENT = 1000000
PACKED = 2 * EMBED_DIM
BPW = BATCH // NUM_WORKERS
CHUNK = 128
NCHUNK = BPW // CHUNK
IDX_ROWS_PER_W = BPW // CHUNK

PACK_R = 2000


def _pack_body(in_ref, out_ref):
    x = in_ref[...]
    out_ref[...] = jnp.concatenate([x[:PACK_R // 2], x[PACK_R // 2:]], axis=1)


def _pack_pairs(ent):
    return pl.pallas_call(
        _pack_body,
        grid=(NUM_ENT // PACK_R,),
        in_specs=[pl.BlockSpec((PACK_R, EMBED_DIM), lambda i: (i, 0))],
        out_specs=pl.BlockSpec((PACK_R // 2, PACKED), lambda i: (i, 0)),
        out_shape=jax.ShapeDtypeStruct((NUM_ENT // 2, PACKED), jnp.float32),
    )(ent)


def _sc_body(hp_h, rp_h, tp_h, par_h, ent_h, relemb_h, out_h,
             hidx, ridx, tidx, par, hrows, rrows, trows, outv, sem):
    wid = lax.axis_index("s") * NUM_CORES + lax.axis_index("c")
    rbase = wid * IDX_ROWS_PER_W

    pltpu.sync_copy(hp_h.at[pl.ds(rbase, IDX_ROWS_PER_W)], hidx)
    pltpu.sync_copy(rp_h.at[pl.ds(rbase, IDX_ROWS_PER_W)], ridx)
    pltpu.sync_copy(tp_h.at[pl.ds(rbase, IDX_ROWS_PER_W)], tidx)
    pltpu.sync_copy(par_h.at[:, pl.ds(rbase, IDX_ROWS_PER_W)], par)

    lanes = lax.iota(jnp.int32, 16)
    dnums = lax.GatherDimensionNumbers(
        offset_dims=(), collapsed_slice_dims=(0,), start_index_map=(0,))

    def lane_sum(v):
        for s in (8, 4, 2, 1):
            perm = lax.gather(
                v, (lanes ^ s)[:, None], dimension_numbers=dnums,
                slice_sizes=(1,),
                mode=lax.GatherScatterMode.PROMISE_IN_BOUNDS)
            v = v + perm
        return v

    for j in range(NCHUNK):
        pltpu.async_copy(ent_h.at[hidx.at[j]], hrows, sem)
        pltpu.async_copy(relemb_h.at[ridx.at[j]], rrows, sem)
        cp = pltpu.async_copy(ent_h.at[tidx.at[j]], trows, sem)
        pltpu.make_async_copy(ent_h.at[hidx.at[j]], hrows, sem).wait()
        pltpu.make_async_copy(relemb_h.at[ridx.at[j]], rrows, sem).wait()
        cp.wait()

        def group(g, carry, j=j):
            base = g * 16
            pvh = par[0, j, pl.ds(base, 16)]
            pvt = par[1, j, pl.ds(base, 16)]
            scores = jnp.zeros((16,), jnp.float32)
            for k in range(16):
                b = base + k
                oh = pvh[k] * EMBED_DIM
                ot = pvt[k] * EMBED_DIM
                acc = (hrows[b, pl.ds(oh, 16)] * rrows[b, pl.ds(0, 16)]
                       * trows[b, pl.ds(ot, 16)])
                for c in range(1, EMBED_DIM // 16):
                    acc = acc + (hrows[b, pl.ds(oh + c * 16, 16)]
                                 * rrows[b, pl.ds(c * 16, 16)]
                                 * trows[b, pl.ds(ot + c * 16, 16)])
                scores = jnp.where(lanes == k, lane_sum(acc), scores)
            outv[pl.ds(j * CHUNK + base, 16)] = scores
            return carry

        lax.fori_loop(0, CHUNK // 16, group, 0)

    pltpu.sync_copy(outv, out_h.at[pl.ds(wid * BPW, BPW)])


def _packed_coords(e):
    row = (e // PACK_R) * (PACK_R // 2) + e % (PACK_R // 2)
    half = (e // (PACK_R // 2)) % 2
    return row, half


@jax.jit
def kernel(head, relation, tail, entity_embeddings, relation_embeddings):
    head = head.astype(jnp.int32)
    relation = relation.astype(jnp.int32)
    tail = tail.astype(jnp.int32)
    hrow, hpar = _packed_coords(head)
    trow, tpar = _packed_coords(tail)
    hp = hrow.reshape(BATCH // CHUNK, CHUNK)
    rp = relation.reshape(BATCH // CHUNK, CHUNK)
    tp = trow.reshape(BATCH // CHUNK, CHUNK)
    par = jnp.stack([hpar.reshape(BATCH // CHUNK, CHUNK),
                     tpar.reshape(BATCH // CHUNK, CHUNK)])

    ent2 = _pack_pairs(entity_embeddings)

    mesh = plsc.VectorSubcoreMesh(core_axis_name="c", subcore_axis_name="s")
    run = functools.partial(
        pl.kernel,
        mesh=mesh,
        compiler_params=pltpu.CompilerParams(use_tc_tiling_on_sc=False),
        out_type=jax.ShapeDtypeStruct((BATCH,), jnp.float32),
        scratch_types=[
            pltpu.VMEM((IDX_ROWS_PER_W, CHUNK), jnp.int32),
            pltpu.VMEM((IDX_ROWS_PER_W, CHUNK), jnp.int32),
            pltpu.VMEM((IDX_ROWS_PER_W, CHUNK), jnp.int32),
            pltpu.VMEM((2, IDX_ROWS_PER_W, CHUNK), jnp.int32),
            pltpu.VMEM((CHUNK, PACKED), jnp.float32),
            pltpu.VMEM((CHUNK, EMBED_DIM), jnp.float32),
            pltpu.VMEM((CHUNK, PACKED), jnp.float32),
            pltpu.VMEM((BPW,), jnp.float32),
            pltpu.SemaphoreType.DMA,
        ],
    )(_sc_body)
    return run(hp, rp, tp, par, ent2, relation_embeddings)


# speedup vs baseline: 1.3044x; 1.3044x over previous
import functools

import jax
import jax.numpy as jnp
from jax import lax
from jax.experimental import pallas as pl
from jax.experimental.pallas import tpu as pltpu
from jax.experimental.pallas import tpu_sc as plsc

NUM_CORES = 2
NUM_SUBCORES = 16
NUM_WORKERS = NUM_CORES * NUM_SUBCORES
BATCH = 16384
EMBED_DIM = 64
BPW = BATCH // NUM_WORKERS
CHUNK = 128
NCHUNK = BPW // CHUNK
IDX_ROWS_PER_W = BPW // CHUNK


def _sc_body(head_h, rel_h, tail_h, ent_h, relemb_h, out_h,
             hidx, ridx, tidx, hrows, rrows, trows, outv, sem):
    wid = lax.axis_index("s") * NUM_CORES + lax.axis_index("c")
    rbase = wid * IDX_ROWS_PER_W

    pltpu.sync_copy(head_h.at[pl.ds(rbase, IDX_ROWS_PER_W)], hidx)
    pltpu.sync_copy(rel_h.at[pl.ds(rbase, IDX_ROWS_PER_W)], ridx)
    pltpu.sync_copy(tail_h.at[pl.ds(rbase, IDX_ROWS_PER_W)], tidx)

    cps = []
    for j in range(NCHUNK):
        cps.append(pltpu.async_copy(
            ent_h.at[hidx.at[j]], hrows.at[pl.ds(j * CHUNK, CHUNK)], sem))
        cps.append(pltpu.async_copy(
            relemb_h.at[ridx.at[j]], rrows.at[pl.ds(j * CHUNK, CHUNK)], sem))
        cps.append(pltpu.async_copy(
            ent_h.at[tidx.at[j]], trows.at[pl.ds(j * CHUNK, CHUNK)], sem))
    for cp in cps:
        cp.wait()

    lanes = lax.iota(jnp.int32, 16)
    dnums = lax.GatherDimensionNumbers(
        offset_dims=(), collapsed_slice_dims=(0,), start_index_map=(0,))

    def lane_sum(v):
        for s in (8, 4, 2, 1):
            perm = lax.gather(
                v, (lanes ^ s)[:, None], dimension_numbers=dnums,
                slice_sizes=(1,),
                mode=lax.GatherScatterMode.PROMISE_IN_BOUNDS)
            v = v + perm
        return v

    def group(g, carry):
        base = g * 16
        scores = jnp.zeros((16,), jnp.float32)
        for j in range(16):
            b = base + j
            acc = (hrows[b, pl.ds(0, 16)] * rrows[b, pl.ds(0, 16)]
                   * trows[b, pl.ds(0, 16)])
            for c in range(1, EMBED_DIM // 16):
                acc = acc + (hrows[b, pl.ds(c * 16, 16)]
                             * rrows[b, pl.ds(c * 16, 16)]
                             * trows[b, pl.ds(c * 16, 16)])
            scores = jnp.where(lanes == j, lane_sum(acc), scores)
        outv[pl.ds(base, 16)] = scores
        return carry

    lax.fori_loop(0, BPW // 16, group, 0)

    pltpu.sync_copy(outv, out_h.at[pl.ds(wid * BPW, BPW)])


@jax.jit
def kernel(head, relation, tail, entity_embeddings, relation_embeddings):
    h = head.astype(jnp.int32).reshape(BATCH // CHUNK, CHUNK)
    r = relation.astype(jnp.int32).reshape(BATCH // CHUNK, CHUNK)
    t = tail.astype(jnp.int32).reshape(BATCH // CHUNK, CHUNK)

    mesh = plsc.VectorSubcoreMesh(core_axis_name="c", subcore_axis_name="s")
    run = functools.partial(
        pl.kernel,
        mesh=mesh,
        compiler_params=pltpu.CompilerParams(use_tc_tiling_on_sc=False),
        out_type=jax.ShapeDtypeStruct((BATCH,), jnp.float32),
        scratch_types=[
            pltpu.VMEM((IDX_ROWS_PER_W, CHUNK), jnp.int32),
            pltpu.VMEM((IDX_ROWS_PER_W, CHUNK), jnp.int32),
            pltpu.VMEM((IDX_ROWS_PER_W, CHUNK), jnp.int32),
            pltpu.VMEM((BPW, EMBED_DIM), jnp.float32),
            pltpu.VMEM((BPW, EMBED_DIM), jnp.float32),
            pltpu.VMEM((BPW, EMBED_DIM), jnp.float32),
            pltpu.VMEM((BPW,), jnp.float32),
            pltpu.SemaphoreType.DMA,
        ],
    )(_sc_body)
    return run(h, r, t, entity_embeddings, relation_embeddings)
